# baseline (speedup 1.0000x reference)
import functools

import jax
import jax.numpy as jnp
from jax import lax
from jax.experimental import pallas as pl
from jax.experimental.pallas import tpu as pltpu

_BEAM = 4
_VOCAB = 100000
_PAD_V = 100096
_NEG = -1e30


def _topk_rp_body(logits_ref, rp_ref, pen_ref, idx_ref, prob_ref, rp_out_ref):
    x = logits_ref[...]
    m = jnp.max(x)
    lse = jnp.log(jnp.sum(jnp.exp(x - m))) + m
    cols = lax.broadcasted_iota(jnp.int32, (1, _PAD_V), 1)
    vals = []
    idxs = []
    xc = x
    for _ in range(_BEAM):
        mk = jnp.max(xc)
        ik = jnp.min(jnp.where(xc == mk, cols, _PAD_V))
        vals.append(mk)
        idxs.append(ik)
        xc = jnp.where(cols == ik, _NEG, xc)
    for k in range(_BEAM):
        idx_ref[k, 0] = idxs[k]
        prob_ref[k, 0] = vals[k] - lse
    rcols = lax.broadcasted_iota(jnp.int32, (1, _VOCAB), 1)
    mask = (
        (rcols == idxs[0]) | (rcols == idxs[1]) | (rcols == idxs[2]) | (rcols == idxs[3])
    )
    p = pen_ref[0]
    rp_out_ref[...] = rp_ref[...] * jnp.where(mask, p, jnp.float32(1.0))


def _topk_rp(logits, rp, pen):
    logits_pad = jnp.pad(logits, ((0, 0), (0, _PAD_V - _VOCAB)), constant_values=_NEG)
    return pl.pallas_call(
        _topk_rp_body,
        in_specs=[
            pl.BlockSpec(memory_space=pltpu.VMEM),
            pl.BlockSpec(memory_space=pltpu.VMEM),
            pl.BlockSpec(memory_space=pltpu.SMEM),
        ],
        out_specs=[
            pl.BlockSpec(memory_space=pltpu.SMEM),
            pl.BlockSpec(memory_space=pltpu.SMEM),
            pl.BlockSpec(memory_space=pltpu.VMEM),
        ],
        out_shape=[
            jax.ShapeDtypeStruct((_BEAM, 1), jnp.int32),
            jax.ShapeDtypeStruct((_BEAM, 1), jnp.float32),
            jax.ShapeDtypeStruct((_BEAM, _VOCAB), jnp.float32),
        ],
    )(logits_pad, rp, pen)


def _bcast_body(in_ref, out_ref):
    out_ref[...] = jnp.broadcast_to(in_ref[...], out_ref.shape)


def _bcast_kv(kv, chunk=256):
    _, h, s, d = kv.shape
    grid = s // chunk
    return pl.pallas_call(
        _bcast_body,
        grid=(grid,),
        in_specs=[
            pl.BlockSpec((1, h, chunk, d), lambda j: (0, 0, j, 0)),
        ],
        out_specs=pl.BlockSpec((_BEAM, h, chunk, d), lambda j: (0, 0, j, 0)),
        out_shape=jax.ShapeDtypeStruct((_BEAM, h, s, d), kv.dtype),
    )(kv)


def kernel(kv_0, kv_1, kv_2, kv_3, kv_4, kv_5, kv_6, kv_7, kv_8, kv_9, kv_10,
           kv_11, logits, save_id, repeat_penality, penality_value, beam_size):
    kvs = [kv_0, kv_1, kv_2, kv_3, kv_4, kv_5, kv_6, kv_7, kv_8, kv_9, kv_10, kv_11]
    tiled = [_bcast_kv(kv) for kv in kvs]
    idx, prob, rp_out = _topk_rp(logits, repeat_penality, penality_value)
    save_id_out = jnp.concatenate([save_id, idx], axis=-1)
    batch_indices = jnp.arange(_BEAM, dtype=jnp.int32) + (beam_size - _BEAM)
    max_logits_idx = idx[0]
    return (*tiled, idx, save_id_out, rp_out, prob, batch_indices, max_logits_idx)


# speedup vs baseline: 1.0195x; 1.0195x over previous
import jax
import jax.numpy as jnp
from jax import lax
from jax.experimental import pallas as pl
from jax.experimental.pallas import tpu as pltpu

_BEAM = 4
_VOCAB = 100000
_PAD_V = 100096
_NEG = -1e30
_HEADS = 16
_SEQ = 2048
_HDIM = 64
_CH = 4
_NCH = _HEADS // _CH
_NKV = 12


def _topk_compute(logits_ref, rp_ref, pen_ref, idx_ref, prob_ref, rp_out_ref):
    x = logits_ref[...]
    m = jnp.max(x)
    lse = jnp.log(jnp.sum(jnp.exp(x - m))) + m
    cols = lax.broadcasted_iota(jnp.int32, (1, _PAD_V), 1)
    vals = []
    idxs = []
    xc = x
    for _ in range(_BEAM):
        mk = jnp.max(xc)
        ik = jnp.min(jnp.where(xc == mk, cols, _PAD_V))
        vals.append(mk)
        idxs.append(ik)
        xc = jnp.where(cols == ik, _NEG, xc)
    for k in range(_BEAM):
        idx_ref[k, 0] = idxs[k]
        prob_ref[k, 0] = vals[k] - lse
    rcols = lax.broadcasted_iota(jnp.int32, (1, _VOCAB), 1)
    mask = (
        (rcols == idxs[0]) | (rcols == idxs[1]) | (rcols == idxs[2]) | (rcols == idxs[3])
    )
    p = pen_ref[0]
    rp_out_ref[...] = rp_ref[...] * jnp.where(mask, p, jnp.float32(1.0))


def _body(*refs):
    kv_in = refs[:_NKV]
    logits_ref, rp_ref, pen_ref = refs[_NKV:_NKV + 3]
    kv_out = refs[_NKV + 3:2 * _NKV + 3]
    idx_ref, prob_ref, rp_out_ref = refs[2 * _NKV + 3:2 * _NKV + 6]
    bufs, rsem, wsem = refs[2 * _NKV + 6:]

    total = _NKV * _NCH

    def read_for(t, slot):
        i, c = divmod(t, _NCH)
        return pltpu.make_async_copy(
            kv_in[i].at[0, pl.ds(c * _CH, _CH)], bufs.at[slot], rsem.at[slot])

    def writes_for(t, slot):
        i, c = divmod(t, _NCH)
        return [
            pltpu.make_async_copy(
                bufs.at[slot], kv_out[i].at[b, pl.ds(c * _CH, _CH)], wsem.at[slot])
            for b in range(_BEAM)
        ]

    rd = {0: read_for(0, 0), 1: read_for(1, 1)}
    rd[0].start()
    rd[1].start()

    _topk_compute(logits_ref, rp_ref, pen_ref, idx_ref, prob_ref, rp_out_ref)

    pending = {}
    for t in range(total):
        slot = t % 2
        if t >= 2:
            for w in pending[t - 2]:
                w.wait()
            rd[t] = read_for(t, slot)
            rd[t].start()
        rd[t].wait()
        ws = writes_for(t, slot)
        for w in ws:
            w.start()
        pending[t] = ws
    for t in (total - 2, total - 1):
        for w in pending[t]:
            w.wait()


def kernel(kv_0, kv_1, kv_2, kv_3, kv_4, kv_5, kv_6, kv_7, kv_8, kv_9, kv_10,
           kv_11, logits, save_id, repeat_penality, penality_value, beam_size):
    kvs = [kv_0, kv_1, kv_2, kv_3, kv_4, kv_5, kv_6, kv_7, kv_8, kv_9, kv_10, kv_11]
    logits_pad = jnp.pad(logits, ((0, 0), (0, _PAD_V - _VOCAB)), constant_values=_NEG)

    hbm = pl.BlockSpec(memory_space=pltpu.MemorySpace.HBM)
    vmem = pl.BlockSpec(memory_space=pltpu.MemorySpace.VMEM)
    smem = pl.BlockSpec(memory_space=pltpu.MemorySpace.SMEM)

    out = pl.pallas_call(
        _body,
        in_specs=[hbm] * _NKV + [vmem, vmem, smem],
        out_specs=[hbm] * _NKV + [smem, smem, vmem],
        out_shape=(
            [jax.ShapeDtypeStruct((_BEAM, _HEADS, _SEQ, _HDIM), jnp.float32)] * _NKV
            + [
                jax.ShapeDtypeStruct((_BEAM, 1), jnp.int32),
                jax.ShapeDtypeStruct((_BEAM, 1), jnp.float32),
                jax.ShapeDtypeStruct((_BEAM, _VOCAB), jnp.float32),
            ]
        ),
        scratch_shapes=[
            pltpu.VMEM((2, _CH, _SEQ, _HDIM), jnp.float32),
            pltpu.SemaphoreType.DMA((2,)),
            pltpu.SemaphoreType.DMA((2,)),
        ],
    )(*kvs, logits_pad, repeat_penality, penality_value)

    tiled = out[:_NKV]
    idx, prob, rp_out = out[_NKV:]
    save_id_out = jnp.concatenate([save_id, idx], axis=-1)
    batch_indices = jnp.arange(_BEAM, dtype=jnp.int32) + (beam_size - _BEAM)
    max_logits_idx = idx[0]
    return (*tiled, idx, save_id_out, rp_out, prob, batch_indices, max_logits_idx)


# speedup vs baseline: 1.0332x; 1.0134x over previous
import jax
import jax.numpy as jnp
from jax import lax
from jax.experimental import pallas as pl
from jax.experimental.pallas import tpu as pltpu

_BEAM = 4
_VOCAB = 100000
_PAD_V = 100096
_NEG = -1e30
_HEADS = 16
_SEQ = 2048
_HDIM = 64
_CH = 16
_NCH = _HEADS // _CH
_NKV = 12


def _topk_compute(logits_ref, rp_ref, pen_ref, idx_ref, prob_ref, rp_out_ref):
    x = logits_ref[...]
    m = jnp.max(x)
    lse = jnp.log(jnp.sum(jnp.exp(x - m))) + m
    cols = lax.broadcasted_iota(jnp.int32, (1, _PAD_V), 1)
    vals = []
    idxs = []
    xc = x
    for _ in range(_BEAM):
        mk = jnp.max(xc)
        ik = jnp.min(jnp.where(xc == mk, cols, _PAD_V))
        vals.append(mk)
        idxs.append(ik)
        xc = jnp.where(cols == ik, _NEG, xc)
    for k in range(_BEAM):
        idx_ref[k, 0] = idxs[k]
        prob_ref[k, 0] = vals[k] - lse
    rcols = lax.broadcasted_iota(jnp.int32, (1, _VOCAB), 1)
    mask = (
        (rcols == idxs[0]) | (rcols == idxs[1]) | (rcols == idxs[2]) | (rcols == idxs[3])
    )
    p = pen_ref[0]
    rp_out_ref[...] = rp_ref[...] * jnp.where(mask, p, jnp.float32(1.0))


def _body(*refs):
    kv_in = refs[:_NKV]
    logits_ref, rp_ref, pen_ref = refs[_NKV:_NKV + 3]
    kv_out = refs[_NKV + 3:2 * _NKV + 3]
    idx_ref, prob_ref, rp_out_ref = refs[2 * _NKV + 3:2 * _NKV + 6]
    bufs, rsem, wsem = refs[2 * _NKV + 6:]

    total = _NKV * _NCH

    def read_for(t, slot):
        i, c = divmod(t, _NCH)
        return pltpu.make_async_copy(
            kv_in[i].at[0, pl.ds(c * _CH, _CH)], bufs.at[slot], rsem.at[slot])

    def writes_for(t, slot):
        i, c = divmod(t, _NCH)
        return [
            pltpu.make_async_copy(
                bufs.at[slot], kv_out[i].at[b, pl.ds(c * _CH, _CH)], wsem.at[slot])
            for b in range(_BEAM)
        ]

    rd = {0: read_for(0, 0), 1: read_for(1, 1)}
    rd[0].start()
    rd[1].start()

    _topk_compute(logits_ref, rp_ref, pen_ref, idx_ref, prob_ref, rp_out_ref)

    pending = {}
    for t in range(total):
        slot = t % 2
        if t >= 2:
            for w in pending[t - 2]:
                w.wait()
            rd[t] = read_for(t, slot)
            rd[t].start()
        rd[t].wait()
        ws = writes_for(t, slot)
        for w in ws:
            w.start()
        pending[t] = ws
    for t in (total - 2, total - 1):
        for w in pending[t]:
            w.wait()


def kernel(kv_0, kv_1, kv_2, kv_3, kv_4, kv_5, kv_6, kv_7, kv_8, kv_9, kv_10,
           kv_11, logits, save_id, repeat_penality, penality_value, beam_size):
    kvs = [kv_0, kv_1, kv_2, kv_3, kv_4, kv_5, kv_6, kv_7, kv_8, kv_9, kv_10, kv_11]
    logits_pad = jnp.pad(logits, ((0, 0), (0, _PAD_V - _VOCAB)), constant_values=_NEG)

    hbm = pl.BlockSpec(memory_space=pltpu.MemorySpace.HBM)
    vmem = pl.BlockSpec(memory_space=pltpu.MemorySpace.VMEM)
    smem = pl.BlockSpec(memory_space=pltpu.MemorySpace.SMEM)

    out = pl.pallas_call(
        _body,
        in_specs=[hbm] * _NKV + [vmem, vmem, smem],
        out_specs=[hbm] * _NKV + [smem, smem, vmem],
        out_shape=(
            [jax.ShapeDtypeStruct((_BEAM, _HEADS, _SEQ, _HDIM), jnp.float32)] * _NKV
            + [
                jax.ShapeDtypeStruct((_BEAM, 1), jnp.int32),
                jax.ShapeDtypeStruct((_BEAM, 1), jnp.float32),
                jax.ShapeDtypeStruct((_BEAM, _VOCAB), jnp.float32),
            ]
        ),
        scratch_shapes=[
            pltpu.VMEM((2, _CH, _SEQ, _HDIM), jnp.float32),
            pltpu.SemaphoreType.DMA((2,)),
            pltpu.SemaphoreType.DMA((2,)),
        ],
    )(*kvs, logits_pad, repeat_penality, penality_value)

    tiled = out[:_NKV]
    idx, prob, rp_out = out[_NKV:]
    save_id_out = jnp.concatenate([save_id, idx], axis=-1)
    batch_indices = jnp.arange(_BEAM, dtype=jnp.int32) + (beam_size - _BEAM)
    max_logits_idx = idx[0]
    return (*tiled, idx, save_id_out, rp_out, prob, batch_indices, max_logits_idx)
